# Initial kernel scaffold; baseline (speedup 1.0000x reference)
import jax
import jax.numpy as jnp
from jax.experimental import pallas as pl


def kernel(x, gW1, gb1, gW2, gb2, gW3, gb3, eW1, eb1, eW2, eb2, eW3, eb3):
    raise NotImplementedError("write your pallas kernel here")


# speedup vs baseline: 1.2257x; 1.2257x over previous
import functools

import jax
import jax.numpy as jnp
from jax.experimental import pallas as pl

B = 16384
D_IN = 17
D_OUT = 6
N_CLUSTERS = 4
H_EXP = 8
BLK = 2048


def _fused_kernel(x_ref, gW1_ref, gb1_ref, gW2_ref, gb2_ref, gW3_ref, gb3_ref,
                  e1_ref, eb1_ref, e2_ref, eb2_ref, e3_ref, eb3_ref,
                  pred_ref, logits_ref):
    f32 = jnp.float32
    x = x_ref[...]

    h = jnp.maximum(jnp.dot(x, gW1_ref[...], preferred_element_type=f32) + gb1_ref[...], 0.0)
    h = jnp.maximum(jnp.dot(h, gW2_ref[...], preferred_element_type=f32) + gb2_ref[...], 0.0)
    logits = jnp.dot(h, gW3_ref[...], preferred_element_type=f32) + gb3_ref[...]
    logits_ref[...] = logits

    blk = logits.shape[0]
    m = jnp.max(logits, axis=1, keepdims=True)
    iota4 = jax.lax.broadcasted_iota(jnp.int32, (blk, N_CLUSTERS), 1)
    sel = jnp.min(jnp.where(logits == m, iota4, N_CLUSTERS), axis=1, keepdims=True)

    h1 = jnp.maximum(jnp.dot(x, e1_ref[...], preferred_element_type=f32) + eb1_ref[...], 0.0)
    h2 = jnp.maximum(jnp.dot(h1, e2_ref[...], preferred_element_type=f32) + eb2_ref[...], 0.0)

    group = jax.lax.broadcasted_iota(jnp.int32, (blk, N_CLUSTERS * H_EXP), 1) // H_EXP
    h2m = jnp.where(group == sel, h2, 0.0)
    onehot = (iota4 == sel).astype(f32)
    pred_ref[...] = (jnp.dot(h2m, e3_ref[...], preferred_element_type=f32)
                     + jnp.dot(onehot, eb3_ref[...], preferred_element_type=f32))


@functools.partial(jax.jit, static_argnames=())
def kernel(x, gW1, gb1, gW2, gb2, gW3, gb3, eW1, eb1, eW2, eb2, eW3, eb3):
    e1 = eW1.transpose(1, 0, 2).reshape(D_IN, N_CLUSTERS * H_EXP)
    eb1f = eb1.reshape(1, N_CLUSTERS * H_EXP)
    e2 = jax.scipy.linalg.block_diag(*[eW2[i] for i in range(N_CLUSTERS)])
    eb2f = eb2.reshape(1, N_CLUSTERS * H_EXP)
    e3 = eW3.reshape(N_CLUSTERS * H_EXP, D_OUT)

    grid = (B // BLK,)
    row_spec = lambda shape: pl.BlockSpec((BLK, shape[1]), lambda i: (i, 0))
    full_spec = lambda a: pl.BlockSpec(a.shape, lambda i: (0,) * a.ndim)

    gb1r, gb2r, gb3r = gb1.reshape(1, -1), gb2.reshape(1, -1), gb3.reshape(1, -1)
    ins = (x, gW1, gb1r, gW2, gb2r, gW3, gb3r, e1, eb1f, e2, eb2f, e3, eb3)
    in_specs = [row_spec(x.shape)] + [full_spec(a) for a in ins[1:]]

    pred, logits = pl.pallas_call(
        _fused_kernel,
        grid=grid,
        in_specs=in_specs,
        out_specs=[
            pl.BlockSpec((BLK, D_OUT), lambda i: (i, 0)),
            pl.BlockSpec((BLK, N_CLUSTERS), lambda i: (i, 0)),
        ],
        out_shape=[
            jax.ShapeDtypeStruct((B, D_OUT), jnp.float32),
            jax.ShapeDtypeStruct((B, N_CLUSTERS), jnp.float32),
        ],
    )(*ins)
    return pred, logits
